# Initial kernel scaffold; baseline (speedup 1.0000x reference)
import jax
import jax.numpy as jnp
from jax.experimental import pallas as pl


def kernel(x, edge_index, batch, W1, b1, W2, b2, W_stab, b_stab, W_sol, b_sol, W_agg, b_agg):
    raise NotImplementedError("write your pallas kernel here")


# speedup vs baseline: 10.1006x; 10.1006x over previous
import functools

import jax
import jax.numpy as jnp
from jax import lax
from jax.experimental import pallas as pl
from jax.experimental.pallas import tpu as pltpu
from jax.experimental.pallas import tpu_sc as plsc

N_NODES = 10000
N_PAD = 10240
N_EDGES = 160000
E_PAD = 163840
D_IN = 256
HID = 512
CW = 128
NCH = HID // CW
NG = 64
NT = 16
RPT = N_PAD // NT
EB = 128
DEG_B = E_PAD // (2 * NT * EB)
PROP_B = E_PAD // (NT * EB)
ROW_BLK = 256
N_BLKS = N_PAD // ROW_BLK

_mesh = plsc.VectorSubcoreMesh(core_axis_name="c", subcore_axis_name="s")


@functools.partial(
    pl.kernel,
    out_type=jax.ShapeDtypeStruct((2, N_PAD, 16), jnp.float32),
    mesh=_mesh,
    scratch_types=[
        pltpu.VMEM((128, 16), jnp.float32),
        pltpu.VMEM((DEG_B, 128), jnp.int32),
        pltpu.VMEM_SHARED((N_PAD, 16), jnp.float32),
    ],
)
def _sc_degree(ones_hbm, dst_hbm, out_hbm, ubuf, idxv, acc):
    c = lax.axis_index("c")
    s = lax.axis_index("s")
    row0 = s * RPT
    for k in range(RPT // 128):
        sl = pl.ds(row0 + k * 128, 128)
        pltpu.sync_copy(ones_hbm.at[sl], ubuf)
        pltpu.sync_copy(ubuf, acc.at[sl])
    w = c * NT + s
    pltpu.sync_copy(dst_hbm.at[pl.ds(w * DEG_B, DEG_B)], idxv)
    plsc.subcore_barrier()

    def body(j, carry):
        pltpu.sync_copy(ubuf, acc.at[idxv.at[j]], add=True)
        return carry

    lax.fori_loop(0, DEG_B, body, 0)
    plsc.subcore_barrier()
    for k in range(RPT // 128):
        sl = pl.ds(row0 + k * 128, 128)
        pltpu.sync_copy(acc.at[sl], ubuf)
        pltpu.sync_copy(ubuf, out_hbm.at[c, sl])


@functools.partial(
    pl.kernel,
    out_type=[jax.ShapeDtypeStruct((N_PAD, CW), jnp.float32)] * NCH,
    mesh=_mesh,
    scratch_types=[
        pltpu.VMEM((PROP_B, 128), jnp.int32),
        pltpu.VMEM((PROP_B, 128), jnp.int32),
        pltpu.VMEM((EB, CW), jnp.float32),
        pltpu.VMEM_SHARED((N_PAD, CW), jnp.float32),
        pltpu.SemaphoreType.DMA,
    ],
)
def _sc_propagate(t0, t1, t2, t3, src_hbm, dst_hbm,
                  o0, o1, o2, o3, srcv, dstv, gbuf, acc, sem):
    c = lax.axis_index("c")
    s = lax.axis_index("s")
    row0 = s * RPT
    pltpu.sync_copy(src_hbm.at[pl.ds(s * PROP_B, PROP_B)], srcv)
    pltpu.sync_copy(dst_hbm.at[pl.ds(s * PROP_B, PROP_B)], dstv)
    t_refs = (t0, t1, t2, t3)
    o_refs = (o0, o1, o2, o3)
    for cc in range(2):
        @pl.when(c == cc)
        def _():
            for lc in range(2):
                t_ref = t_refs[cc * 2 + lc]
                o_ref = o_refs[cc * 2 + lc]
                for k in range(RPT // 128):
                    sl = pl.ds(row0 + k * 128, 128)
                    pltpu.sync_copy(t_ref.at[sl], gbuf)
                    pltpu.sync_copy(gbuf, acc.at[sl])
                plsc.subcore_barrier()

                def body(j, carry):
                    pltpu.async_copy(t_ref.at[srcv.at[j]], gbuf, sem).wait()
                    pltpu.sync_copy(gbuf, acc.at[dstv.at[j]], add=True)
                    return carry

                lax.fori_loop(0, PROP_B, body, 0)
                plsc.subcore_barrier()
                for k in range(RPT // 128):
                    sl = pl.ds(row0 + k * 128, 128)
                    pltpu.sync_copy(acc.at[sl], gbuf)
                    pltpu.sync_copy(gbuf, o_ref.at[sl])


def _tc1_body(hist_ref, x_ref, w1_ref, dis_ref, t0_ref, t1_ref, t2_ref, t3_ref):
    deg = hist_ref[0, :, 0:1] + hist_ref[1, :, 0:1] - 1.0
    dis = 1.0 / jnp.sqrt(deg)
    dis_ref[...] = dis
    t = jnp.dot(x_ref[...], w1_ref[...],
                preferred_element_type=jnp.float32) * dis
    t0_ref[...] = t[:, 0:128]
    t1_ref[...] = t[:, 128:256]
    t2_ref[...] = t[:, 256:384]
    t3_ref[...] = t[:, 384:512]


def _tc2_body(a0, a1, a2, a3, dis_ref, b_ref, w2_ref,
              t0_ref, t1_ref, t2_ref, t3_ref):
    accv = jnp.concatenate([a0[...], a1[...], a2[...], a3[...]], axis=1)
    dis = dis_ref[...]
    h = jnp.maximum(accv * dis + b_ref[...], 0.0)
    t = jnp.dot(h, w2_ref[...], preferred_element_type=jnp.float32) * dis
    t0_ref[...] = t[:, 0:128]
    t1_ref[...] = t[:, 128:256]
    t2_ref[...] = t[:, 256:384]
    t3_ref[...] = t[:, 384:512]


def _tc3_body(a0, a1, a2, a3, dis_ref, b_ref, batch_ref, wcat_ref, bcat_ref,
              out_ref, gsum, gcnt):
    i = pl.program_id(0)

    @pl.when(i == 0)
    def _():
        gsum[...] = jnp.zeros_like(gsum)
        gcnt[...] = jnp.zeros_like(gcnt)

    accv = jnp.concatenate([a0[...], a1[...], a2[...], a3[...]], axis=1)
    h = jnp.maximum(accv * dis_ref[...] + b_ref[...], 0.0)
    gids = lax.broadcasted_iota(jnp.int32, (NG, ROW_BLK), 0)
    mask = (gids == batch_ref[...]).astype(jnp.float32)
    gsum[...] += jnp.dot(mask, h, preferred_element_type=jnp.float32)
    gcnt[...] += jnp.sum(mask, axis=1, keepdims=True)

    @pl.when(i == N_BLKS - 1)
    def _():
        g = gsum[...] / jnp.maximum(gcnt[...], 1.0)
        out_ref[...] = jnp.dot(g, wcat_ref[...],
                               preferred_element_type=jnp.float32) + bcat_ref[...]


_tc1 = pl.pallas_call(
    _tc1_body,
    grid=(N_BLKS,),
    in_specs=[
        pl.BlockSpec((2, ROW_BLK, 16), lambda i: (0, i, 0)),
        pl.BlockSpec((ROW_BLK, D_IN), lambda i: (i, 0)),
        pl.BlockSpec((D_IN, HID), lambda i: (0, 0)),
    ],
    out_specs=[pl.BlockSpec((ROW_BLK, 1), lambda i: (i, 0))]
    + [pl.BlockSpec((ROW_BLK, CW), lambda i: (i, 0))] * NCH,
    out_shape=[jax.ShapeDtypeStruct((N_PAD, 1), jnp.float32)]
    + [jax.ShapeDtypeStruct((N_PAD, CW), jnp.float32)] * NCH,
)

_tc2 = pl.pallas_call(
    _tc2_body,
    grid=(N_BLKS,),
    in_specs=[pl.BlockSpec((ROW_BLK, CW), lambda i: (i, 0))] * NCH
    + [
        pl.BlockSpec((ROW_BLK, 1), lambda i: (i, 0)),
        pl.BlockSpec((1, HID), lambda i: (0, 0)),
        pl.BlockSpec((HID, HID), lambda i: (0, 0)),
    ],
    out_specs=[pl.BlockSpec((ROW_BLK, CW), lambda i: (i, 0))] * NCH,
    out_shape=[jax.ShapeDtypeStruct((N_PAD, CW), jnp.float32)] * NCH,
)

_tc3 = pl.pallas_call(
    _tc3_body,
    grid=(N_BLKS,),
    in_specs=[pl.BlockSpec((ROW_BLK, CW), lambda i: (i, 0))] * NCH
    + [
        pl.BlockSpec((ROW_BLK, 1), lambda i: (i, 0)),
        pl.BlockSpec((1, HID), lambda i: (0, 0)),
        pl.BlockSpec((1, ROW_BLK), lambda i: (0, i)),
        pl.BlockSpec((HID, 128), lambda i: (0, 0)),
        pl.BlockSpec((1, 128), lambda i: (0, 0)),
    ],
    out_specs=pl.BlockSpec((NG, 128), lambda i: (0, 0)),
    out_shape=jax.ShapeDtypeStruct((NG, 128), jnp.float32),
    scratch_shapes=[
        pltpu.VMEM((NG, HID), jnp.float32),
        pltpu.VMEM((NG, 1), jnp.float32),
    ],
)


def kernel(x, edge_index, batch, W1, b1, W2, b2,
           W_stab, b_stab, W_sol, b_sol, W_agg, b_agg):
    src = edge_index[0].astype(jnp.int32)
    dst = edge_index[1].astype(jnp.int32)
    npad = E_PAD - N_EDGES
    pad_ar = jnp.arange(npad, dtype=jnp.int32)
    src_p = jnp.concatenate([src, pad_ar % N_NODES]).reshape(E_PAD // 128, 128)
    dst_p = jnp.concatenate(
        [dst, N_NODES + pad_ar % (N_PAD - N_NODES)]).reshape(E_PAD // 128, 128)
    x_p = jnp.concatenate(
        [x, jnp.zeros((N_PAD - N_NODES, D_IN), jnp.float32)])
    batch_p = jnp.concatenate(
        [batch.astype(jnp.int32),
         jnp.full((N_PAD - N_NODES,), NG, jnp.int32)]).reshape(1, N_PAD)
    ones16 = jnp.ones((N_PAD, 16), jnp.float32)
    w_cat = jnp.pad(jnp.concatenate([W_stab, W_sol, W_agg], axis=1),
                    ((0, 0), (0, 125)))
    b_cat = jnp.pad(jnp.concatenate([b_stab, b_sol, b_agg]),
                    (0, 125)).reshape(1, 128)

    hist = _sc_degree(ones16, dst_p)
    dis, t0, t1, t2, t3 = _tc1(hist, x_p, W1)
    a0, a1, a2, a3 = _sc_propagate(t0, t1, t2, t3, src_p, dst_p)
    u0, u1, u2, u3 = _tc2(a0, a1, a2, a3, dis, b1.reshape(1, HID), W2)
    c0, c1, c2, c3 = _sc_propagate(u0, u1, u2, u3, src_p, dst_p)
    out = _tc3(c0, c1, c2, c3, dis, b2.reshape(1, HID), batch_p, w_cat, b_cat)
    return (out[:, 0:1], out[:, 1:2], out[:, 2:3])


# speedup vs baseline: 14.2689x; 1.4127x over previous
import functools

import jax
import jax.numpy as jnp
from jax import lax
from jax.experimental import pallas as pl
from jax.experimental.pallas import tpu as pltpu
from jax.experimental.pallas import tpu_sc as plsc

N_NODES = 10000
N_PAD = 10240
N_EDGES = 160000
E_PAD = 163840
D_IN = 256
HID = 512
CW = 128
NCH = HID // CW
NG = 64
NT = 16
RPT = N_PAD // NT
EB = 128
DEG_B = E_PAD // (2 * NT * EB)
PROP_B = E_PAD // (NT * EB)
ROW_BLK = 256
N_BLKS = N_PAD // ROW_BLK

_mesh = plsc.VectorSubcoreMesh(core_axis_name="c", subcore_axis_name="s")


@functools.partial(
    pl.kernel,
    out_type=jax.ShapeDtypeStruct((2, N_PAD, 16), jnp.float32),
    mesh=_mesh,
    scratch_types=[
        pltpu.VMEM((128, 16), jnp.float32),
        pltpu.VMEM((DEG_B, 128), jnp.int32),
        pltpu.VMEM_SHARED((N_PAD, 16), jnp.float32),
    ],
)
def _sc_degree(ones_hbm, dst_hbm, out_hbm, ubuf, idxv, acc):
    c = lax.axis_index("c")
    s = lax.axis_index("s")
    row0 = s * RPT
    for k in range(RPT // 128):
        sl = pl.ds(row0 + k * 128, 128)
        pltpu.sync_copy(ones_hbm.at[sl], ubuf)
        pltpu.sync_copy(ubuf, acc.at[sl])
    w = c * NT + s
    pltpu.sync_copy(dst_hbm.at[pl.ds(w * DEG_B, DEG_B)], idxv)
    plsc.subcore_barrier()

    def body(j, carry):
        pltpu.sync_copy(ubuf, acc.at[idxv.at[j]], add=True)
        return carry

    lax.fori_loop(0, DEG_B, body, 0)
    plsc.subcore_barrier()
    for k in range(RPT // 128):
        sl = pl.ds(row0 + k * 128, 128)
        pltpu.sync_copy(acc.at[sl], ubuf)
        pltpu.sync_copy(ubuf, out_hbm.at[c, sl])


@functools.partial(
    pl.kernel,
    out_type=[jax.ShapeDtypeStruct((N_PAD, CW), jnp.float32)] * NCH,
    mesh=_mesh,
    scratch_types=[
        pltpu.VMEM((PROP_B // 2, 128), jnp.int32),
        pltpu.VMEM((PROP_B // 2, 128), jnp.int32),
        pltpu.VMEM((EB, CW), jnp.float32),
        pltpu.VMEM((EB, CW), jnp.float32),
        pltpu.VMEM_SHARED((N_PAD, CW), jnp.float32),
        pltpu.SemaphoreType.DMA,
        pltpu.SemaphoreType.DMA,
    ],
)
def _sc_propagate(t0, t1, t2, t3, src_hbm, dst_hbm,
                  o0, o1, o2, o3, srcv, dstv, ga, gb, acc, sema, semb):
    c = lax.axis_index("c")
    s = lax.axis_index("s")
    row0 = s * RPT
    hb = PROP_B // 2
    t_refs = (t0, t1, t2, t3)
    o_refs = (o0, o1, o2, o3)
    for cc in range(2):
        @pl.when(c == cc)
        def _():
            for lc in range(2):
                t_ref = t_refs[cc * 2 + lc]
                o_ref = o_refs[cc * 2 + lc]
                for k in range(RPT // 128):
                    sl = pl.ds(row0 + k * 128, 128)
                    pltpu.sync_copy(t_ref.at[sl], ga)
                    pltpu.sync_copy(ga, acc.at[sl])
                plsc.subcore_barrier()

                for half in range(2):
                    e0 = s * PROP_B + half * hb
                    pltpu.sync_copy(src_hbm.at[pl.ds(e0, hb)], srcv)
                    pltpu.sync_copy(dst_hbm.at[pl.ds(e0, hb)], dstv)
                    pltpu.async_copy(t_ref.at[srcv.at[0]], ga, sema)

                    def body(jj, carry):
                        j0 = jj * 2
                        j1 = j0 + 1
                        pltpu.async_copy(t_ref.at[srcv.at[j1]], gb, semb)
                        pltpu.make_async_copy(
                            t_ref.at[srcv.at[j0]], ga, sema).wait()
                        pltpu.sync_copy(ga, acc.at[dstv.at[j0]], add=True)
                        j2 = lax.rem(j0 + 2, hb)
                        pltpu.async_copy(t_ref.at[srcv.at[j2]], ga, sema)
                        pltpu.make_async_copy(
                            t_ref.at[srcv.at[j1]], gb, semb).wait()
                        pltpu.sync_copy(gb, acc.at[dstv.at[j1]], add=True)
                        return carry

                    lax.fori_loop(0, hb // 2, body, 0)
                    pltpu.make_async_copy(t_ref.at[srcv.at[0]], ga, sema).wait()
                plsc.subcore_barrier()
                for k in range(RPT // 128):
                    sl = pl.ds(row0 + k * 128, 128)
                    pltpu.sync_copy(acc.at[sl], ga)
                    pltpu.sync_copy(ga, o_ref.at[sl])


def _tc1_body(hist_ref, x_ref, w1_ref, dis_ref, t0_ref, t1_ref, t2_ref, t3_ref):
    deg = hist_ref[0, :, 0:1] + hist_ref[1, :, 0:1] - 1.0
    dis = 1.0 / jnp.sqrt(deg)
    dis_ref[...] = dis
    t = jnp.dot(x_ref[...], w1_ref[...],
                preferred_element_type=jnp.float32) * dis
    t0_ref[...] = t[:, 0:128]
    t1_ref[...] = t[:, 128:256]
    t2_ref[...] = t[:, 256:384]
    t3_ref[...] = t[:, 384:512]


def _tc2_body(a0, a1, a2, a3, dis_ref, b_ref, w2_ref,
              t0_ref, t1_ref, t2_ref, t3_ref):
    accv = jnp.concatenate([a0[...], a1[...], a2[...], a3[...]], axis=1)
    dis = dis_ref[...]
    h = jnp.maximum(accv * dis + b_ref[...], 0.0)
    t = jnp.dot(h, w2_ref[...], preferred_element_type=jnp.float32) * dis
    t0_ref[...] = t[:, 0:128]
    t1_ref[...] = t[:, 128:256]
    t2_ref[...] = t[:, 256:384]
    t3_ref[...] = t[:, 384:512]


def _tc3_body(a0, a1, a2, a3, dis_ref, b_ref, batch_ref, wcat_ref, bcat_ref,
              out_ref, gsum, gcnt):
    i = pl.program_id(0)

    @pl.when(i == 0)
    def _():
        gsum[...] = jnp.zeros_like(gsum)
        gcnt[...] = jnp.zeros_like(gcnt)

    accv = jnp.concatenate([a0[...], a1[...], a2[...], a3[...]], axis=1)
    h = jnp.maximum(accv * dis_ref[...] + b_ref[...], 0.0)
    gids = lax.broadcasted_iota(jnp.int32, (NG, ROW_BLK), 0)
    mask = (gids == batch_ref[...]).astype(jnp.float32)
    gsum[...] += jnp.dot(mask, h, preferred_element_type=jnp.float32)
    gcnt[...] += jnp.sum(mask, axis=1, keepdims=True)

    @pl.when(i == N_BLKS - 1)
    def _():
        g = gsum[...] / jnp.maximum(gcnt[...], 1.0)
        out_ref[...] = jnp.dot(g, wcat_ref[...],
                               preferred_element_type=jnp.float32) + bcat_ref[...]


_tc1 = pl.pallas_call(
    _tc1_body,
    grid=(N_BLKS,),
    in_specs=[
        pl.BlockSpec((2, ROW_BLK, 16), lambda i: (0, i, 0)),
        pl.BlockSpec((ROW_BLK, D_IN), lambda i: (i, 0)),
        pl.BlockSpec((D_IN, HID), lambda i: (0, 0)),
    ],
    out_specs=[pl.BlockSpec((ROW_BLK, 1), lambda i: (i, 0))]
    + [pl.BlockSpec((ROW_BLK, CW), lambda i: (i, 0))] * NCH,
    out_shape=[jax.ShapeDtypeStruct((N_PAD, 1), jnp.float32)]
    + [jax.ShapeDtypeStruct((N_PAD, CW), jnp.float32)] * NCH,
)

_tc2 = pl.pallas_call(
    _tc2_body,
    grid=(N_BLKS,),
    in_specs=[pl.BlockSpec((ROW_BLK, CW), lambda i: (i, 0))] * NCH
    + [
        pl.BlockSpec((ROW_BLK, 1), lambda i: (i, 0)),
        pl.BlockSpec((1, HID), lambda i: (0, 0)),
        pl.BlockSpec((HID, HID), lambda i: (0, 0)),
    ],
    out_specs=[pl.BlockSpec((ROW_BLK, CW), lambda i: (i, 0))] * NCH,
    out_shape=[jax.ShapeDtypeStruct((N_PAD, CW), jnp.float32)] * NCH,
)

_tc3 = pl.pallas_call(
    _tc3_body,
    grid=(N_BLKS,),
    in_specs=[pl.BlockSpec((ROW_BLK, CW), lambda i: (i, 0))] * NCH
    + [
        pl.BlockSpec((ROW_BLK, 1), lambda i: (i, 0)),
        pl.BlockSpec((1, HID), lambda i: (0, 0)),
        pl.BlockSpec((1, ROW_BLK), lambda i: (0, i)),
        pl.BlockSpec((HID, 128), lambda i: (0, 0)),
        pl.BlockSpec((1, 128), lambda i: (0, 0)),
    ],
    out_specs=pl.BlockSpec((NG, 128), lambda i: (0, 0)),
    out_shape=jax.ShapeDtypeStruct((NG, 128), jnp.float32),
    scratch_shapes=[
        pltpu.VMEM((NG, HID), jnp.float32),
        pltpu.VMEM((NG, 1), jnp.float32),
    ],
)


def kernel(x, edge_index, batch, W1, b1, W2, b2,
           W_stab, b_stab, W_sol, b_sol, W_agg, b_agg):
    src = edge_index[0].astype(jnp.int32)
    dst = edge_index[1].astype(jnp.int32)
    npad = E_PAD - N_EDGES
    pad_ar = jnp.arange(npad, dtype=jnp.int32)
    src_p = jnp.concatenate([src, pad_ar % N_NODES]).reshape(E_PAD // 128, 128)
    dst_p = jnp.concatenate(
        [dst, N_NODES + pad_ar % (N_PAD - N_NODES)]).reshape(E_PAD // 128, 128)
    x_p = jnp.concatenate(
        [x, jnp.zeros((N_PAD - N_NODES, D_IN), jnp.float32)])
    batch_p = jnp.concatenate(
        [batch.astype(jnp.int32),
         jnp.full((N_PAD - N_NODES,), NG, jnp.int32)]).reshape(1, N_PAD)
    ones16 = jnp.ones((N_PAD, 16), jnp.float32)
    w_cat = jnp.pad(jnp.concatenate([W_stab, W_sol, W_agg], axis=1),
                    ((0, 0), (0, 125)))
    b_cat = jnp.pad(jnp.concatenate([b_stab, b_sol, b_agg]),
                    (0, 125)).reshape(1, 128)

    hist = _sc_degree(ones16, dst_p)
    dis, t0, t1, t2, t3 = _tc1(hist, x_p, W1)
    a0, a1, a2, a3 = _sc_propagate(t0, t1, t2, t3, src_p, dst_p)
    u0, u1, u2, u3 = _tc2(a0, a1, a2, a3, dis, b1.reshape(1, HID), W2)
    c0, c1, c2, c3 = _sc_propagate(u0, u1, u2, u3, src_p, dst_p)
    out = _tc3(c0, c1, c2, c3, dis, b2.reshape(1, HID), batch_p, w_cat, b_cat)
    return (out[:, 0:1], out[:, 1:2], out[:, 2:3])
